# baseline (speedup 1.0000x reference)
import functools

import jax
import jax.numpy as jnp
from jax import lax
from jax.experimental import pallas as pl
from jax.experimental.pallas import tpu as pltpu
from jax.experimental.pallas import tpu_sc as plsc

NC = 2
NS = 16
LANES = 16
G = 2


@functools.lru_cache(maxsize=None)
def _make_sc_scatter(N, E, D):
    NW = NC * NS
    assert E % NW == 0, E
    e_per_w = E // NW
    CH = 96
    n_ch = e_per_w // CH
    CT = e_per_w - n_ch * CH
    assert CT % 8 == 0 and CT % LANES == 0, CT
    n_pad = -(-N // (NS * 16)) * (NS * 16)
    rows_per_tile = n_pad // NS
    z_rows = G * n_pad // NS
    z_full, z_rem = divmod(z_rows, CH)
    o_full, o_rem = divmod(rows_per_tile, CH)
    assert z_rem % 16 == 0 and o_rem % 16 == 0

    mesh = plsc.VectorSubcoreMesh(core_axis_name="c", subcore_axis_name="s")

    @functools.partial(
        pl.kernel,
        out_type=(
            jax.ShapeDtypeStruct((n_pad, D), jnp.float32),
            jax.ShapeDtypeStruct((n_pad, D), jnp.float32),
        ),
        mesh=mesh,
        compiler_params=pltpu.CompilerParams(
            use_tc_tiling_on_sc=False, needs_layout_passes=False),
        scratch_types=[
            pltpu.VMEM_SHARED((G * n_pad, D), jnp.bfloat16),
            [pltpu.VMEM((CH, D), jnp.float32) for _ in range(2)],
            [pltpu.VMEM((CH, D), jnp.bfloat16) for _ in range(2)],
            [pltpu.VMEM((CH,), jnp.int32) for _ in range(2)],
            [pltpu.VMEM((CH,), jnp.int32) for _ in range(2)],
            [pltpu.SemaphoreType.DMA for _ in range(2)],
            [pltpu.SemaphoreType.DMA for _ in range(2)],
            pltpu.VMEM((CT, D), jnp.float32),
            pltpu.VMEM((CT, D), jnp.bfloat16),
            pltpu.VMEM((CT,), jnp.int32),
            pltpu.VMEM((CT,), jnp.int32),
        ],
    )
    def sc_scatter(e_hbm, src_hbm, dst_hbm, out0_hbm, out1_hbm, acc,
                   e_v, e_bf, si_v, di_v, lsem, ssem, et_v, et_bf, sit_v, dit_v):
        c = lax.axis_index("c")
        s = lax.axis_index("s")
        wid = c * NS + s
        goff = jnp.where(s < NS // G, 0, n_pad).astype(jnp.int32)

        e_bf0 = e_bf[0]

        def zrow(i, carry):
            for j in range(D // 32):
                e_bf0[i, pl.ds(j * 32, 32)] = jnp.zeros((32,), jnp.bfloat16)
            return carry

        lax.fori_loop(0, CH, zrow, 0)
        zbase = s * z_rows

        def zcopy(i, carry):
            pltpu.sync_copy(e_bf0, acc.at[pl.ds(zbase + i * CH, CH)])
            return carry

        lax.fori_loop(0, z_full, zcopy, 0)
        if z_rem:
            pltpu.sync_copy(e_bf0.at[pl.ds(0, z_rem)],
                            acc.at[pl.ds(zbase + z_full * CH, z_rem)])
        plsc.subcore_barrier()

        base0 = wid * e_per_w

        def start_load(b, chunk):
            base = base0 + chunk * CH
            pltpu.async_copy(src_hbm.at[pl.ds(base, CH)], si_v[b], lsem[b])
            pltpu.async_copy(dst_hbm.at[pl.ds(base, CH)], di_v[b], lsem[b])
            pltpu.async_copy(e_hbm.at[pl.ds(base, CH)], e_v[b], lsem[b])

        def wait_load(b):
            pltpu.make_async_copy(src_hbm.at[pl.ds(base0, CH)], si_v[b], lsem[b]).wait()
            pltpu.make_async_copy(dst_hbm.at[pl.ds(base0, CH)], di_v[b], lsem[b]).wait()
            pltpu.make_async_copy(e_hbm.at[pl.ds(base0, CH)], e_v[b], lsem[b]).wait()

        def convert(b):
            src_ref, dst_ref = e_v[b], e_bf[b]

            def crow(i, carry):
                for j in range(D // 32):
                    x = src_ref[i, pl.ds(j * 32, 16)]
                    y = src_ref[i, pl.ds(j * 32 + 16, 16)]
                    dst_ref[i, pl.ds(j * 32, 32)] = plsc.pack(
                        x, y, format=plsc.PackFormat.INTERLEAVED)
                return carry

            lax.fori_loop(0, CH, crow, 0)
            for ref in (si_v[b], di_v[b]):
                for k in range(CH // LANES):
                    ref[pl.ds(k * LANES, LANES)] = ref[pl.ds(k * LANES, LANES)] + goff

        def start_scatter(b):
            pltpu.async_copy(e_bf[b], acc.at[si_v[b]], ssem[b], add=True)
            pltpu.async_copy(e_bf[b], acc.at[di_v[b]], ssem[b], add=True)

        def wait_scatter(b):
            pltpu.make_async_copy(e_bf[b], acc.at[si_v[b]], ssem[b]).wait()
            pltpu.make_async_copy(e_bf[b], acc.at[di_v[b]], ssem[b]).wait()

        def slot(i, first, last):
            b = i % 2
            wait_load(b)
            convert(b)
            start_scatter(b)
            if not first:
                wait_scatter(1 - b)
            if not last:
                start_load(1 - b, i + 1)

        start_load(0, 0)
        slot(0, True, False)
        n_main = (n_ch - 2) // 2

        def step(g, carry):
            for k in range(2):
                i = 1 + 2 * g + k
                b = (1 + k) % 2
                wait_load(b)
                convert(b)
                start_scatter(b)
                wait_scatter(1 - b)
                start_load(1 - b, i + 1)
            return carry

        lax.fori_loop(0, n_main, step, 0)
        for i in range(1 + n_main * 2, n_ch):
            slot(i, False, i + 1 >= n_ch)
        if CT:
            tbase = base0 + n_ch * CH
            pltpu.sync_copy(src_hbm.at[pl.ds(tbase, CT)], sit_v)
            pltpu.sync_copy(dst_hbm.at[pl.ds(tbase, CT)], dit_v)
            pltpu.sync_copy(e_hbm.at[pl.ds(tbase, CT)], et_v)

            def trow(i, carry):
                for j in range(D // 32):
                    x = et_v[i, pl.ds(j * 32, 16)]
                    y = et_v[i, pl.ds(j * 32 + 16, 16)]
                    et_bf[i, pl.ds(j * 32, 32)] = plsc.pack(
                        x, y, format=plsc.PackFormat.INTERLEAVED)
                return carry

            lax.fori_loop(0, CT, trow, 0)
            for ref in (sit_v, dit_v):
                for k in range(CT // LANES):
                    ref[pl.ds(k * LANES, LANES)] = ref[pl.ds(k * LANES, LANES)] + goff
        wait_scatter((n_ch - 1) % 2)
        if CT:
            pltpu.sync_copy(et_bf, acc.at[sit_v], add=True)
            pltpu.sync_copy(et_bf, acc.at[dit_v], add=True)
        plsc.subcore_barrier()

        obase = s * rows_per_tile
        out_acc = e_v[0]

        def emit(r0, rows):
            pltpu.sync_copy(acc.at[pl.ds(r0, rows)], e_bf[0].at[pl.ds(0, rows)])
            pltpu.sync_copy(acc.at[pl.ds(n_pad + r0, rows)], e_bf[1].at[pl.ds(0, rows)])

            def orow(i, carry):
                for j in range(D // 32):
                    a0, a1 = plsc.unpack(e_bf[0][i, pl.ds(j * 32, 32)],
                                         format=plsc.PackFormat.INTERLEAVED)
                    b0, b1 = plsc.unpack(e_bf[1][i, pl.ds(j * 32, 32)],
                                         format=plsc.PackFormat.INTERLEAVED)
                    out_acc[i, pl.ds(j * 32, 16)] = a0 + b0
                    out_acc[i, pl.ds(j * 32 + 16, 16)] = a1 + b1
                return carry

            lax.fori_loop(0, rows, orow, 0)

            @pl.when(c == 0)
            def _():
                pltpu.sync_copy(out_acc.at[pl.ds(0, rows)], out0_hbm.at[pl.ds(r0, rows)])

            @pl.when(c == 1)
            def _():
                pltpu.sync_copy(out_acc.at[pl.ds(0, rows)], out1_hbm.at[pl.ds(r0, rows)])

        for jj in range(o_full):
            emit(obase + jj * CH, CH)
        if o_rem:
            emit(obase + o_full * CH, o_rem)

    return sc_scatter


@functools.lru_cache(maxsize=None)
def _make_tc_matmul(N, D):
    BN = 2000
    assert N % BN == 0

    def mm(v_ref, p0_ref, p1_ref, w0_ref, w1_ref, b_ref, o_ref):
        v1 = p0_ref[...] + p1_ref[...]
        o_ref[...] = (
            jnp.dot(v_ref[...], w0_ref[...], preferred_element_type=jnp.float32)
            + jnp.dot(v1, w1_ref[...], preferred_element_type=jnp.float32)
            + b_ref[...]
        )

    return pl.pallas_call(
        mm,
        grid=(N // BN,),
        in_specs=[
            pl.BlockSpec((BN, D), lambda i: (i, 0)),
            pl.BlockSpec((BN, D), lambda i: (i, 0)),
            pl.BlockSpec((BN, D), lambda i: (i, 0)),
            pl.BlockSpec((D, D), lambda i: (0, 0)),
            pl.BlockSpec((D, D), lambda i: (0, 0)),
            pl.BlockSpec((1, D), lambda i: (0, 0)),
        ],
        out_specs=pl.BlockSpec((BN, D), lambda i: (i, 0)),
        out_shape=jax.ShapeDtypeStruct((N, D), jnp.float32),
    )


def kernel(e, v, edges, W, b):
    E, D = e.shape
    N = v.shape[0]
    src = edges[:, 0]
    dst = edges[:, 1]
    p0, p1 = _make_sc_scatter(N, E, D)(e, src, dst)
    mm = _make_tc_matmul(N, D)
    return mm(v, p0, p1, W[:D], W[D:], b.reshape(1, D))


# speedup vs baseline: 2.0179x; 2.0179x over previous
import functools

import jax
import jax.numpy as jnp
from jax import lax
from jax.experimental import pallas as pl
from jax.experimental.pallas import tpu as pltpu
from jax.experimental.pallas import tpu_sc as plsc

NC = 2
NS = 16
LANES = 16
NBUF = 2


@functools.lru_cache(maxsize=None)
def _make_sc_scatter(N, E, D):
    NW = NC * NS
    assert E % NW == 0, E
    e_per_w = E // NW
    CH = 128
    n_ch = e_per_w // CH
    CT = e_per_w - n_ch * CH
    assert CT % 8 == 0, CT
    ZCH = 64
    n_pad = -(-N // (NS * CH)) * (NS * CH)
    z_per_tile = n_pad // NS
    n_z = z_per_tile // ZCH

    mesh = plsc.VectorSubcoreMesh(core_axis_name="c", subcore_axis_name="s")

    @functools.partial(
        pl.kernel,
        out_type=(
            jax.ShapeDtypeStruct((n_pad, D), jnp.float32),
            jax.ShapeDtypeStruct((n_pad, D), jnp.float32),
        ),
        mesh=mesh,
        scratch_types=[
            pltpu.VMEM_SHARED((n_pad, D), jnp.float32),
            [pltpu.VMEM((CH, D), jnp.float32) for _ in range(NBUF)],
            [pltpu.VMEM((CH,), jnp.int32) for _ in range(NBUF)],
            [pltpu.VMEM((CH,), jnp.int32) for _ in range(NBUF)],
            [pltpu.SemaphoreType.DMA for _ in range(NBUF)],
            [pltpu.SemaphoreType.DMA for _ in range(NBUF)],
            pltpu.VMEM((CT, D), jnp.float32),
            pltpu.VMEM((CT,), jnp.int32),
            pltpu.VMEM((CT,), jnp.int32),
            pltpu.VMEM((ZCH, D), jnp.float32),
            pltpu.SemaphoreType.DMA,
        ],
    )
    def sc_scatter(e_hbm, src_hbm, dst_hbm, out0_hbm, out1_hbm, acc,
                   e_v, si_v, di_v, lsem, ssem, et_v, sit_v, dit_v, zbuf, zsem):
        c = lax.axis_index("c")
        s = lax.axis_index("s")
        wid = c * NS + s
        base0 = wid * e_per_w

        def start_load(b, chunk):
            base = base0 + chunk * CH
            pltpu.async_copy(src_hbm.at[pl.ds(base, CH)], si_v[b], lsem[b])
            pltpu.async_copy(dst_hbm.at[pl.ds(base, CH)], di_v[b], lsem[b])
            pltpu.async_copy(e_hbm.at[pl.ds(base, CH)], e_v[b], lsem[b])

        def wait_load(b):
            pltpu.make_async_copy(src_hbm.at[pl.ds(base0, CH)], si_v[b], lsem[b]).wait()
            pltpu.make_async_copy(dst_hbm.at[pl.ds(base0, CH)], di_v[b], lsem[b]).wait()
            pltpu.make_async_copy(e_hbm.at[pl.ds(base0, CH)], e_v[b], lsem[b]).wait()

        def start_scatter(b):
            pltpu.async_copy(e_v[b], acc.at[si_v[b]], ssem[b], add=True)
            pltpu.async_copy(e_v[b], acc.at[di_v[b]], ssem[b], add=True)

        def wait_scatter(b):
            pltpu.make_async_copy(e_v[b], acc.at[si_v[b]], ssem[b]).wait()
            pltpu.make_async_copy(e_v[b], acc.at[di_v[b]], ssem[b]).wait()

        start_load(0, 0)
        start_load(1, 1)

        def zrow(i, carry):
            for j in range(D // LANES):
                zbuf[i, pl.ds(j * LANES, LANES)] = jnp.zeros((LANES,), jnp.float32)
            return carry

        lax.fori_loop(0, ZCH, zrow, 0)
        zbase = s * z_per_tile

        def zstart(i, carry):
            pltpu.async_copy(zbuf, acc.at[pl.ds(zbase + i * ZCH, ZCH)], zsem)
            return carry

        lax.fori_loop(0, n_z, zstart, 0)

        def zwait(i, carry):
            pltpu.make_async_copy(zbuf, acc.at[pl.ds(zbase, ZCH)], zsem).wait()
            return carry

        lax.fori_loop(0, n_z, zwait, 0)
        plsc.subcore_barrier()

        def slot(i, first, last):
            b = i % 2
            wait_load(b)
            start_scatter(b)
            if not first:
                wait_scatter(1 - b)
            if not last:
                start_load(1 - b, i + 1)

        wait_load(0)
        start_scatter(0)

        n_main = (n_ch - 1 - 1) // 2

        def step(g, carry):
            for k in range(2):
                i = 1 + 2 * g + k
                b = (1 + k) % 2
                wait_load(b)
                start_scatter(b)
                wait_scatter(1 - b)
                start_load(1 - b, i + 1)
            return carry

        lax.fori_loop(0, n_main, step, 0)
        for i in range(1 + n_main * 2, n_ch):
            slot(i, False, i + 1 >= n_ch)
        if CT:
            tbase = base0 + n_ch * CH
            pltpu.sync_copy(src_hbm.at[pl.ds(tbase, CT)], sit_v)
            pltpu.sync_copy(dst_hbm.at[pl.ds(tbase, CT)], dit_v)
            pltpu.sync_copy(e_hbm.at[pl.ds(tbase, CT)], et_v)
        wait_scatter((n_ch - 1) % 2)
        if CT:
            pltpu.sync_copy(et_v, acc.at[sit_v], add=True)
            pltpu.sync_copy(et_v, acc.at[dit_v], add=True)
        plsc.subcore_barrier()

        row_slice = pl.ds(s * z_per_tile, z_per_tile)

        @pl.when(c == 0)
        def _():
            pltpu.sync_copy(acc.at[row_slice], out0_hbm.at[row_slice])

        @pl.when(c == 1)
        def _():
            pltpu.sync_copy(acc.at[row_slice], out1_hbm.at[row_slice])

    return sc_scatter


@functools.lru_cache(maxsize=None)
def _make_tc_matmul(N, D):
    BN = 2000
    assert N % BN == 0

    def mm(v_ref, p0_ref, p1_ref, w0_ref, w1_ref, b_ref, o_ref):
        v1 = p0_ref[...] + p1_ref[...]
        o_ref[...] = (
            jnp.dot(v_ref[...], w0_ref[...], preferred_element_type=jnp.float32)
            + jnp.dot(v1, w1_ref[...], preferred_element_type=jnp.float32)
            + b_ref[...]
        )

    return pl.pallas_call(
        mm,
        grid=(N // BN,),
        in_specs=[
            pl.BlockSpec((BN, D), lambda i: (i, 0)),
            pl.BlockSpec((BN, D), lambda i: (i, 0)),
            pl.BlockSpec((BN, D), lambda i: (i, 0)),
            pl.BlockSpec((D, D), lambda i: (0, 0)),
            pl.BlockSpec((D, D), lambda i: (0, 0)),
            pl.BlockSpec((1, D), lambda i: (0, 0)),
        ],
        out_specs=pl.BlockSpec((BN, D), lambda i: (i, 0)),
        out_shape=jax.ShapeDtypeStruct((N, D), jnp.float32),
    )


def kernel(e, v, edges, W, b):
    E, D = e.shape
    N = v.shape[0]
    src = edges[:, 0]
    dst = edges[:, 1]
    p0, p1 = _make_sc_scatter(N, E, D)(e, src, dst)
    mm = _make_tc_matmul(N, D)
    return mm(v, p0, p1, W[:D], W[D:], b.reshape(1, D))


# speedup vs baseline: 2.0198x; 1.0009x over previous
import functools

import jax
import jax.numpy as jnp
from jax import lax
from jax.experimental import pallas as pl
from jax.experimental.pallas import tpu as pltpu
from jax.experimental.pallas import tpu_sc as plsc

NC = 2
NS = 16
LANES = 16
NBUF = 2


@functools.lru_cache(maxsize=None)
def _make_sc_scatter(N, E, D):
    NW = NC * NS
    assert E % NW == 0, E
    e_per_w = E // NW
    CH = 128
    n_ch = e_per_w // CH
    CT = e_per_w - n_ch * CH
    assert CT % 8 == 0, CT
    ZCH = 64
    n_pad = -(-N // (NS * CH)) * (NS * CH)
    z_per_tile = n_pad // NS
    n_z = z_per_tile // ZCH

    mesh = plsc.VectorSubcoreMesh(core_axis_name="c", subcore_axis_name="s")

    @functools.partial(
        pl.kernel,
        out_type=(
            jax.ShapeDtypeStruct((n_pad, D), jnp.float32),
            jax.ShapeDtypeStruct((n_pad, D), jnp.float32),
        ),
        mesh=mesh,
        scratch_types=[
            pltpu.VMEM_SHARED((n_pad, D), jnp.float32),
            [pltpu.VMEM((CH, D), jnp.float32) for _ in range(NBUF)],
            [pltpu.VMEM((CH,), jnp.int32) for _ in range(NBUF)],
            [pltpu.VMEM((CH,), jnp.int32) for _ in range(NBUF)],
            [pltpu.SemaphoreType.DMA for _ in range(NBUF)],
            [pltpu.SemaphoreType.DMA for _ in range(NBUF)],
            pltpu.VMEM((CT, D), jnp.float32),
            pltpu.VMEM((CT,), jnp.int32),
            pltpu.VMEM((CT,), jnp.int32),
            pltpu.VMEM((ZCH, D), jnp.float32),
            pltpu.SemaphoreType.DMA,
        ],
    )
    def sc_scatter(e_hbm, src_hbm, dst_hbm, out0_hbm, out1_hbm, acc,
                   e_v, si_v, di_v, lsem, ssem, et_v, sit_v, dit_v, zbuf, zsem):
        c = lax.axis_index("c")
        s = lax.axis_index("s")
        wid = c * NS + s
        base0 = wid * e_per_w

        def start_load(b, chunk):
            base = base0 + chunk * CH
            pltpu.async_copy(src_hbm.at[pl.ds(base, CH)], si_v[b], lsem[b])
            pltpu.async_copy(dst_hbm.at[pl.ds(base, CH)], di_v[b], lsem[b])
            pltpu.async_copy(e_hbm.at[pl.ds(base, CH)], e_v[b], lsem[b])

        def wait_load(b):
            pltpu.make_async_copy(src_hbm.at[pl.ds(base0, CH)], si_v[b], lsem[b]).wait()
            pltpu.make_async_copy(dst_hbm.at[pl.ds(base0, CH)], di_v[b], lsem[b]).wait()
            pltpu.make_async_copy(e_hbm.at[pl.ds(base0, CH)], e_v[b], lsem[b]).wait()

        def start_scatter(b):
            pltpu.async_copy(e_v[b], acc.at[si_v[b]], ssem[b], add=True)
            pltpu.async_copy(e_v[b], acc.at[di_v[b]], ssem[b], add=True)

        def wait_scatter(b):
            pltpu.make_async_copy(e_v[b], acc.at[si_v[b]], ssem[b]).wait()
            pltpu.make_async_copy(e_v[b], acc.at[di_v[b]], ssem[b]).wait()

        start_load(0, 0)
        start_load(1, 1)

        def zrow(i, carry):
            for j in range(D // LANES):
                zbuf[i, pl.ds(j * LANES, LANES)] = jnp.zeros((LANES,), jnp.float32)
            return carry

        lax.fori_loop(0, ZCH, zrow, 0)
        zbase = s * z_per_tile

        def zstart(i, carry):
            pltpu.async_copy(zbuf, acc.at[pl.ds(zbase + i * ZCH, ZCH)], zsem)
            return carry

        lax.fori_loop(0, n_z, zstart, 0)

        def zwait(i, carry):
            pltpu.make_async_copy(zbuf, acc.at[pl.ds(zbase, ZCH)], zsem).wait()
            return carry

        lax.fori_loop(0, n_z, zwait, 0)
        plsc.subcore_barrier()

        def slot(i, first, last):
            b = i % 2
            wait_load(b)
            start_scatter(b)
            if not first:
                wait_scatter(1 - b)
            if not last:
                start_load(1 - b, i + 1)

        wait_load(0)
        start_scatter(0)

        n_main = (n_ch - 1 - 1) // 2

        def step(g, carry):
            for k in range(2):
                i = 1 + 2 * g + k
                b = (1 + k) % 2
                wait_load(b)
                start_scatter(b)
                wait_scatter(1 - b)
                start_load(1 - b, i + 1)
            return carry

        lax.fori_loop(0, n_main, step, 0)
        for i in range(1 + n_main * 2, n_ch):
            slot(i, False, i + 1 >= n_ch)
        if CT:
            tbase = base0 + n_ch * CH
            pltpu.sync_copy(src_hbm.at[pl.ds(tbase, CT)], sit_v)
            pltpu.sync_copy(dst_hbm.at[pl.ds(tbase, CT)], dit_v)
            pltpu.sync_copy(e_hbm.at[pl.ds(tbase, CT)], et_v)
        wait_scatter((n_ch - 1) % 2)
        if CT:
            pltpu.sync_copy(et_v, acc.at[sit_v], add=True)
            pltpu.sync_copy(et_v, acc.at[dit_v], add=True)
        plsc.subcore_barrier()

        row_slice = pl.ds(s * z_per_tile, z_per_tile)

        @pl.when(c == 0)
        def _():
            pltpu.sync_copy(acc.at[row_slice], out0_hbm.at[row_slice])

        @pl.when(c == 1)
        def _():
            pltpu.sync_copy(acc.at[row_slice], out1_hbm.at[row_slice])

    return sc_scatter


@functools.lru_cache(maxsize=None)
def _make_tc_matmul_a(N, D):
    BN = 2000
    assert N % BN == 0

    def mm(v_ref, w0_ref, b_ref, o_ref):
        o_ref[...] = (
            jnp.dot(v_ref[...], w0_ref[...], preferred_element_type=jnp.float32)
            + b_ref[...]
        )

    return pl.pallas_call(
        mm,
        grid=(N // BN,),
        in_specs=[
            pl.BlockSpec((BN, D), lambda i: (i, 0)),
            pl.BlockSpec((D, D), lambda i: (0, 0)),
            pl.BlockSpec((1, D), lambda i: (0, 0)),
        ],
        out_specs=pl.BlockSpec((BN, D), lambda i: (i, 0)),
        out_shape=jax.ShapeDtypeStruct((N, D), jnp.float32),
    )


@functools.lru_cache(maxsize=None)
def _make_tc_matmul_b(N, D):
    BN = 2000
    assert N % BN == 0

    def mm(a_ref, p0_ref, p1_ref, w1_ref, o_ref):
        v1 = p0_ref[...] + p1_ref[...]
        o_ref[...] = a_ref[...] + jnp.dot(
            v1, w1_ref[...], preferred_element_type=jnp.float32)

    return pl.pallas_call(
        mm,
        grid=(N // BN,),
        in_specs=[
            pl.BlockSpec((BN, D), lambda i: (i, 0)),
            pl.BlockSpec((BN, D), lambda i: (i, 0)),
            pl.BlockSpec((BN, D), lambda i: (i, 0)),
            pl.BlockSpec((D, D), lambda i: (0, 0)),
        ],
        out_specs=pl.BlockSpec((BN, D), lambda i: (i, 0)),
        out_shape=jax.ShapeDtypeStruct((N, D), jnp.float32),
    )


def kernel(e, v, edges, W, b):
    E, D = e.shape
    N = v.shape[0]
    src = edges[:, 0]
    dst = edges[:, 1]
    p0, p1 = _make_sc_scatter(N, E, D)(e, src, dst)
    mm_a = _make_tc_matmul_a(N, D)(v, W[:D], b.reshape(1, D))
    return _make_tc_matmul_b(N, D)(mm_a, p0, p1, W[D:])
